# Initial kernel scaffold; baseline (speedup 1.0000x reference)
import jax
import jax.numpy as jnp
from jax.experimental import pallas as pl


def kernel(x, edge_index, W_self, W_neigh):
    raise NotImplementedError("write your pallas kernel here")


# speedup vs baseline: 319.3072x; 319.3072x over previous
import functools

import jax
import jax.numpy as jnp
from jax import lax
from jax.experimental import pallas as pl
from jax.experimental.pallas import tpu as pltpu
from jax.experimental.pallas import tpu_sc as plsc

NC = 2
NS = 16
NW = NC * NS
LANES = 16
CHUNK = 8000


@functools.partial(jax.jit, static_argnames=("n_pad",))
def _hop(h_pad, src, dst, n_pad):
  e_tot = src.shape[0]
  ew = e_tot // NW
  n_chunks = ew // CHUNK
  s_sl = n_pad // NS

  mesh = plsc.VectorSubcoreMesh(core_axis_name="c", subcore_axis_name="s")

  @functools.partial(
      pl.kernel,
      out_type=jax.ShapeDtypeStruct((NC * n_pad,), jnp.float32),
      mesh=mesh,
      scratch_types=[
          pltpu.VMEM_SHARED((n_pad,), jnp.float32),
          pltpu.VMEM_SHARED((n_pad,), jnp.float32),
          pltpu.VMEM((CHUNK,), jnp.int32),
          pltpu.VMEM((CHUNK,), jnp.int32),
          pltpu.VMEM((CHUNK,), jnp.float32),
          pltpu.VMEM((s_sl,), jnp.float32),
          pltpu.SemaphoreType.DMA,
      ],
  )
  def hop_kernel(h_hbm, src_hbm, dst_hbm, out_hbm, h_sh, acc_sh, src_v, dst_v,
                 vals_v, zbuf, sem):
    c = lax.axis_index("c")
    s = lax.axis_index("s")
    wid = s * NC + c
    base_n = s * s_sl

    def _zero(i, carry):
      zbuf[pl.ds(i * LANES, LANES)] = jnp.zeros((LANES,), jnp.float32)
      return carry
    lax.fori_loop(0, s_sl // LANES, _zero, 0)
    pltpu.sync_copy(zbuf, acc_sh.at[pl.ds(base_n, s_sl)])
    pltpu.sync_copy(h_hbm.at[pl.ds(base_n, s_sl)], zbuf)
    pltpu.sync_copy(zbuf, h_sh.at[pl.ds(base_n, s_sl)])
    plsc.subcore_barrier()

    base_e = wid * ew
    def _edges(g, carry):
      off = base_e + g * CHUNK
      pltpu.sync_copy(src_hbm.at[pl.ds(off, CHUNK)], src_v)
      pltpu.sync_copy(dst_hbm.at[pl.ds(off, CHUNK)], dst_v)
      pltpu.async_copy(h_sh.at[src_v], vals_v, sem).wait()
      pltpu.sync_copy(vals_v, acc_sh.at[dst_v], add=True)
      return carry
    lax.fori_loop(0, n_chunks, _edges, 0)
    plsc.subcore_barrier()

    pltpu.sync_copy(acc_sh.at[pl.ds(base_n, s_sl)], zbuf)
    pltpu.sync_copy(zbuf, out_hbm.at[pl.ds(c * n_pad + base_n, s_sl)])

  return hop_kernel(h_pad, src, dst)


def kernel(x, edge_index, W_self, W_neigh):
  n, d = x.shape
  num_hop = W_self.shape[0]
  assert d == 1

  n_pad = -(-(n + 1) // (NS * 8)) * (NS * 8)
  h = jnp.zeros((n_pad,), jnp.float32).at[:n].set(x[:, 0])

  e = edge_index.shape[1]
  e_pad = -(-e // (NW * CHUNK)) * (NW * CHUNK)
  if e_pad != e:
    pad = jnp.zeros((2, e_pad - e), jnp.int32).at[1, :].set(n)
    edge_index = jnp.concatenate([edge_index, pad], axis=1)

  src = edge_index[0]
  dst = edge_index[1]
  for i in range(num_hop):
    parts = _hop(h, src, dst, n_pad)
    acc = parts[:n_pad] + parts[n_pad:]
    h = W_self[i, 0, 0] * h + W_neigh[i, 0, 0] * acc
  return h[:n, None]


# speedup vs baseline: 354.6414x; 1.1107x over previous
import functools

import jax
import jax.numpy as jnp
from jax import lax
from jax.experimental import pallas as pl
from jax.experimental.pallas import tpu as pltpu
from jax.experimental.pallas import tpu_sc as plsc

NC = 2
NS = 16
NW = NC * NS
LANES = 16
CHUNK = 4000


@functools.partial(jax.jit, static_argnames=("n_pad",))
def _hop(h_pad, src, dst, n_pad):
  e_tot = src.shape[0]
  ew = e_tot // NW
  n_chunks = ew // CHUNK
  n_pairs = n_chunks // 2
  s_sl = n_pad // NS

  mesh = plsc.VectorSubcoreMesh(core_axis_name="c", subcore_axis_name="s")

  @functools.partial(
      pl.kernel,
      out_type=jax.ShapeDtypeStruct((NC * n_pad,), jnp.float32),
      mesh=mesh,
      compiler_params=pltpu.CompilerParams(needs_layout_passes=False),
      scratch_types=[
          pltpu.VMEM((n_pad,), jnp.float32),
          pltpu.VMEM_SHARED((n_pad,), jnp.float32),
          pltpu.VMEM((CHUNK,), jnp.int32),
          pltpu.VMEM((CHUNK,), jnp.int32),
          pltpu.VMEM((CHUNK,), jnp.int32),
          pltpu.VMEM((CHUNK,), jnp.int32),
          pltpu.VMEM((CHUNK,), jnp.float32),
          pltpu.VMEM((CHUNK,), jnp.float32),
          pltpu.SemaphoreType.DMA,
          pltpu.SemaphoreType.DMA,
      ],
  )
  def hop_kernel(h_hbm, src_hbm, dst_hbm, out_hbm, h_loc, acc_sh,
                 src0, src1, dst0, dst1, vals0, vals1, ld0, ld1):
    c = lax.axis_index("c")
    s = lax.axis_index("s")
    wid = s * NC + c
    base_n = s * s_sl
    base_e = wid * ew

    def _issue(g, sbuf, dbuf, sem):
      off = base_e + lax.rem(g, n_chunks) * CHUNK
      pltpu.async_copy(src_hbm.at[pl.ds(off, CHUNK)], sbuf, sem)
      pltpu.async_copy(dst_hbm.at[pl.ds(off, CHUNK)], dbuf, sem)

    def _wait(sbuf, dbuf, sem):
      pltpu.make_async_copy(src_hbm.at[pl.ds(0, CHUNK)], sbuf, sem).wait()
      pltpu.make_async_copy(dst_hbm.at[pl.ds(0, CHUNK)], dbuf, sem).wait()

    _issue(0, src0, dst0, ld0)

    pltpu.sync_copy(h_hbm, h_loc)

    def _zero(i, carry):
      vals0[pl.ds(i * LANES, LANES)] = jnp.zeros((LANES,), jnp.float32)
      return carry
    lax.fori_loop(0, CHUNK // LANES, _zero, 0)
    off = 0
    while off < s_sl:
      piece = min(CHUNK, s_sl - off)
      pltpu.sync_copy(vals0.at[pl.ds(0, piece)],
                      acc_sh.at[pl.ds(base_n + off, piece)])
      off += piece
    plsc.subcore_barrier()

    def _gather(sbuf, vbuf):
      def body(i, carry):
        idx = sbuf[pl.ds(i * LANES, LANES)]
        vbuf[pl.ds(i * LANES, LANES)] = plsc.load_gather(h_loc, [idx])
        return carry
      lax.fori_loop(0, CHUNK // LANES, body, 0)

    def _pair(p, carry):
      g = 2 * p
      _wait(src0, dst0, ld0)
      _issue(g + 1, src1, dst1, ld1)
      _gather(src0, vals0)
      pltpu.sync_copy(vals0, acc_sh.at[dst0], add=True)
      _wait(src1, dst1, ld1)
      _issue(g + 2, src0, dst0, ld0)
      _gather(src1, vals1)
      pltpu.sync_copy(vals1, acc_sh.at[dst1], add=True)
      return carry
    lax.fori_loop(0, n_pairs, _pair, 0)
    _wait(src0, dst0, ld0)
    plsc.subcore_barrier()

    off = 0
    while off < s_sl:
      piece = min(CHUNK, s_sl - off)
      pltpu.sync_copy(acc_sh.at[pl.ds(base_n + off, piece)],
                      vals0.at[pl.ds(0, piece)])
      pltpu.sync_copy(vals0.at[pl.ds(0, piece)],
                      out_hbm.at[pl.ds(c * n_pad + base_n + off, piece)])
      off += piece

  return hop_kernel(h_pad, src, dst)


def kernel(x, edge_index, W_self, W_neigh):
  n, d = x.shape
  num_hop = W_self.shape[0]
  assert d == 1

  n_pad = -(-(n + 1) // (NS * 8)) * (NS * 8)
  h = jnp.zeros((n_pad,), jnp.float32).at[:n].set(x[:, 0])

  e = edge_index.shape[1]
  e_pad = -(-e // (NW * 2 * CHUNK)) * (NW * 2 * CHUNK)
  if e_pad != e:
    pad = jnp.zeros((2, e_pad - e), jnp.int32).at[1, :].set(n)
    edge_index = jnp.concatenate([edge_index, pad], axis=1)

  src = edge_index[0]
  dst = edge_index[1]
  for i in range(num_hop):
    parts = _hop(h, src, dst, n_pad)
    acc = parts[:n_pad] + parts[n_pad:]
    h = W_self[i, 0, 0] * h + W_neigh[i, 0, 0] * acc
  return h[:n, None]


# speedup vs baseline: 510.8991x; 1.4406x over previous
import functools

import jax
import jax.numpy as jnp
from jax import lax
from jax.experimental import pallas as pl
from jax.experimental.pallas import tpu as pltpu
from jax.experimental.pallas import tpu_sc as plsc

NC = 2
NS = 16
NW = NC * NS
LANES = 16
CHUNK = 2000
NBUF = 4


@functools.partial(jax.jit, static_argnames=("n_pad",))
def _hop(h_pad, src, dst, n_pad):
  e_tot = src.shape[0]
  ew = e_tot // NW
  n_chunks = ew // CHUNK
  n_quads = n_chunks // NBUF
  s_sl = n_pad // NS

  mesh = plsc.VectorSubcoreMesh(core_axis_name="c", subcore_axis_name="s")

  @functools.partial(
      pl.kernel,
      out_type=jax.ShapeDtypeStruct((NC * n_pad,), jnp.float32),
      mesh=mesh,
      compiler_params=pltpu.CompilerParams(needs_layout_passes=False),
      scratch_types=[
          pltpu.VMEM((n_pad,), jnp.float32),
          pltpu.VMEM_SHARED((n_pad,), jnp.float32),
          [pltpu.VMEM((CHUNK,), jnp.int32) for _ in range(NBUF)],
          [pltpu.VMEM((CHUNK,), jnp.int32) for _ in range(NBUF)],
          [pltpu.VMEM((CHUNK,), jnp.float32) for _ in range(NBUF)],
          [pltpu.SemaphoreType.DMA for _ in range(NBUF)],
          [pltpu.SemaphoreType.DMA for _ in range(NBUF)],
          pltpu.SemaphoreType.DMA,
      ],
  )
  def hop_kernel(h_hbm, src_hbm, dst_hbm, out_hbm, h_loc, acc_sh,
                 srcb, dstb, valb, ld, st, ldh):
    c = lax.axis_index("c")
    s = lax.axis_index("s")
    wid = s * NC + c
    base_n = s * s_sl
    base_e = wid * ew

    def _issue_ld(g, b):
      off = base_e + lax.rem(g, n_chunks) * CHUNK
      pltpu.async_copy(src_hbm.at[pl.ds(off, CHUNK)], srcb[b], ld[b])
      pltpu.async_copy(dst_hbm.at[pl.ds(off, CHUNK)], dstb[b], ld[b])

    def _wait_ld(b):
      pltpu.make_async_copy(src_hbm.at[pl.ds(0, CHUNK)], srcb[b], ld[b]).wait()
      pltpu.make_async_copy(dst_hbm.at[pl.ds(0, CHUNK)], dstb[b], ld[b]).wait()

    def _issue_st(b):
      pltpu.async_copy(valb[b], acc_sh.at[dstb[b]], st[b], add=True)

    def _wait_st(b):
      pltpu.make_async_copy(valb[b], acc_sh.at[dstb[b]], st[b]).wait()

    def _gather(b):
      def body(i, carry):
        idx = srcb[b][pl.ds(i * LANES, LANES)]
        valb[b][pl.ds(i * LANES, LANES)] = plsc.load_gather(h_loc, [idx])
        return carry
      lax.fori_loop(0, CHUNK // LANES, body, 0)

    _issue_ld(0, 0)
    _issue_ld(1, 1)
    h_cp = pltpu.async_copy(h_hbm, h_loc, ldh)

    def _zero(i, carry):
      valb[0][pl.ds(i * LANES, LANES)] = jnp.zeros((LANES,), jnp.float32)
      return carry
    lax.fori_loop(0, CHUNK // LANES, _zero, 0)
    off = 0
    while off < s_sl:
      piece = min(CHUNK, s_sl - off)
      pltpu.sync_copy(valb[0].at[pl.ds(0, piece)],
                      acc_sh.at[pl.ds(base_n + off, piece)])
      off += piece
    plsc.subcore_barrier()
    h_cp.wait()

    for g in range(NBUF):
      b = g % NBUF
      _wait_ld(b)
      if g >= 2:
        _wait_st((g + 2) % NBUF)
      _issue_ld(g + 2, (g + 2) % NBUF)
      _gather(b)
      _issue_st(b)

    def _quad(q, carry):
      g0 = q * NBUF
      for b in range(NBUF):
        g = g0 + b
        _wait_ld(b)
        _wait_st((b + 2) % NBUF)
        _issue_ld(g + 2, (b + 2) % NBUF)
        _gather(b)
        _issue_st(b)
      return carry
    lax.fori_loop(1, n_quads, _quad, 0)

    _wait_ld(0)
    _wait_ld(1)
    _wait_st(2)
    _wait_st(3)
    plsc.subcore_barrier()

    off = 0
    while off < s_sl:
      piece = min(CHUNK, s_sl - off)
      pltpu.sync_copy(acc_sh.at[pl.ds(base_n + off, piece)],
                      valb[0].at[pl.ds(0, piece)])
      pltpu.sync_copy(valb[0].at[pl.ds(0, piece)],
                      out_hbm.at[pl.ds(c * n_pad + base_n + off, piece)])
      off += piece

  return hop_kernel(h_pad, src, dst)


def kernel(x, edge_index, W_self, W_neigh):
  n, d = x.shape
  num_hop = W_self.shape[0]
  assert d == 1

  n_pad = -(-(n + 1) // (NS * 8)) * (NS * 8)
  h = jnp.zeros((n_pad,), jnp.float32).at[:n].set(x[:, 0])

  e = edge_index.shape[1]
  e_pad = -(-e // (NW * NBUF * CHUNK)) * (NW * NBUF * CHUNK)
  if e_pad != e:
    pad = jnp.zeros((2, e_pad - e), jnp.int32).at[1, :].set(n)
    edge_index = jnp.concatenate([edge_index, pad], axis=1)

  src = edge_index[0]
  dst = edge_index[1]
  for i in range(num_hop):
    parts = _hop(h, src, dst, n_pad)
    acc = parts[:n_pad] + parts[n_pad:]
    h = W_self[i, 0, 0] * h + W_neigh[i, 0, 0] * acc
  return h[:n, None]


# speedup vs baseline: 547.7916x; 1.0722x over previous
import functools

import jax
import jax.numpy as jnp
from jax import lax
from jax.experimental import pallas as pl
from jax.experimental.pallas import tpu as pltpu
from jax.experimental.pallas import tpu_sc as plsc

NC = 2
NS = 16
NW = NC * NS
LANES = 16
CHUNK = 2000
NBUF = 4


@functools.partial(jax.jit, static_argnames=("n_pad",))
def _hop(h, edges_flat, n_pad):
  n = h.shape[0]
  e_tot = edges_flat.shape[0] // 2
  ew = e_tot // NW
  n_chunks = ew // CHUNK
  n_quads = n_chunks // NBUF
  s_sl = n_pad // NS

  mesh = plsc.VectorSubcoreMesh(core_axis_name="c", subcore_axis_name="s")

  @functools.partial(
      pl.kernel,
      out_type=jax.ShapeDtypeStruct((NC * n_pad,), jnp.float32),
      mesh=mesh,
      compiler_params=pltpu.CompilerParams(needs_layout_passes=False),
      scratch_types=[
          pltpu.VMEM((n,), jnp.float32),
          pltpu.VMEM_SHARED((n_pad,), jnp.float32),
          [pltpu.VMEM((CHUNK,), jnp.int32) for _ in range(NBUF)],
          [pltpu.VMEM((CHUNK,), jnp.int32) for _ in range(NBUF)],
          [pltpu.VMEM((CHUNK,), jnp.float32) for _ in range(NBUF)],
          [pltpu.SemaphoreType.DMA for _ in range(NBUF)],
          [pltpu.SemaphoreType.DMA for _ in range(NBUF)],
          pltpu.SemaphoreType.DMA,
      ],
  )
  def hop_kernel(h_hbm, edges_hbm, out_hbm, h_loc, acc_sh,
                 srcb, dstb, valb, ld, st, ldh):
    c = lax.axis_index("c")
    s = lax.axis_index("s")
    wid = s * NC + c
    base_n = s * s_sl
    base_e = wid * ew

    def _issue_ld(g, b):
      off = base_e + lax.rem(g, n_chunks) * CHUNK
      pltpu.async_copy(edges_hbm.at[pl.ds(off, CHUNK)], srcb[b], ld[b])
      pltpu.async_copy(edges_hbm.at[pl.ds(e_tot + off, CHUNK)], dstb[b], ld[b])

    def _wait_ld(b):
      pltpu.make_async_copy(edges_hbm.at[pl.ds(0, CHUNK)], srcb[b],
                            ld[b]).wait()
      pltpu.make_async_copy(edges_hbm.at[pl.ds(0, CHUNK)], dstb[b],
                            ld[b]).wait()

    def _issue_st(b):
      pltpu.async_copy(valb[b], acc_sh.at[dstb[b]], st[b], add=True)

    def _wait_st(b):
      pltpu.make_async_copy(valb[b], acc_sh.at[dstb[b]], st[b]).wait()

    def _gather(b):
      def body(i, carry):
        idx = srcb[b][pl.ds(i * LANES, LANES)]
        valb[b][pl.ds(i * LANES, LANES)] = plsc.load_gather(h_loc, [idx])
        return carry
      lax.fori_loop(0, CHUNK // LANES, body, 0)

    _issue_ld(0, 0)
    _issue_ld(1, 1)
    h_cp = pltpu.async_copy(h_hbm, h_loc, ldh)

    def _zero(i, carry):
      valb[0][pl.ds(i * LANES, LANES)] = jnp.zeros((LANES,), jnp.float32)
      return carry
    lax.fori_loop(0, CHUNK // LANES, _zero, 0)
    off = 0
    while off < s_sl:
      piece = min(CHUNK, s_sl - off)
      pltpu.sync_copy(valb[0].at[pl.ds(0, piece)],
                      acc_sh.at[pl.ds(base_n + off, piece)])
      off += piece
    plsc.subcore_barrier()
    h_cp.wait()

    for g in range(NBUF):
      b = g % NBUF
      _wait_ld(b)
      if g >= 2:
        _wait_st((g + 2) % NBUF)
      _issue_ld(g + 2, (g + 2) % NBUF)
      _gather(b)
      _issue_st(b)

    def _quad(q, carry):
      g0 = q * NBUF
      for b in range(NBUF):
        g = g0 + b
        _wait_ld(b)
        _wait_st((b + 2) % NBUF)
        _issue_ld(g + 2, (b + 2) % NBUF)
        _gather(b)
        _issue_st(b)
      return carry
    lax.fori_loop(1, n_quads, _quad, 0)

    _wait_ld(0)
    _wait_ld(1)
    _wait_st(2)
    _wait_st(3)
    plsc.subcore_barrier()

    off = 0
    while off < s_sl:
      piece = min(CHUNK, s_sl - off)
      pltpu.sync_copy(acc_sh.at[pl.ds(base_n + off, piece)],
                      valb[0].at[pl.ds(0, piece)])
      pltpu.sync_copy(valb[0].at[pl.ds(0, piece)],
                      out_hbm.at[pl.ds(c * n_pad + base_n + off, piece)])
      off += piece

  return hop_kernel(h, edges_flat)


def kernel(x, edge_index, W_self, W_neigh):
  n, d = x.shape
  num_hop = W_self.shape[0]
  assert d == 1

  n_pad = -(-(n + 1) // (NS * 8)) * (NS * 8)

  e = edge_index.shape[1]
  e_pad = -(-e // (NW * NBUF * CHUNK)) * (NW * NBUF * CHUNK)
  if e_pad != e:
    pad = jnp.zeros((2, e_pad - e), jnp.int32).at[1, :].set(n)
    edge_index = jnp.concatenate([edge_index, pad], axis=1)
  edges_flat = edge_index.reshape(-1)

  h = x[:, 0]
  for i in range(num_hop):
    parts = _hop(h, edges_flat, n_pad)
    acc = parts[:n] + parts[n_pad:n_pad + n]
    h = W_self[i, 0, 0] * h + W_neigh[i, 0, 0] * acc
  return h[:, None]
